# baseline (speedup 1.0000x reference)
import functools

import jax
import jax.numpy as jnp
from jax import lax
from jax.experimental import pallas as pl
from jax.experimental.pallas import tpu as pltpu
from jax.experimental.pallas import tpu_sc as plsc

N = 10000
E = 160000
D = 256
DH = 128
NC = 2
NS = 16
BN = 400
NBLK = N // BN
EPT = E // NS
BATCH = 128
NCH = 80
SLOTS = NCH * BATCH
PADT = SLOTS - EPT
NJ = 16
NACC = N + NJ
WPT = 624
WTAIL = N - NS * WPT

_DN = (((1,), (1,)), ((), ()))


def _front_body(x_ref, wl_ref, wr_ref, b_ref, u_ref, v_ref):
    xb = x_ref[...].astype(jnp.bfloat16)
    u = lax.dot_general(xb, wl_ref[...], _DN, preferred_element_type=jnp.float32)
    u_ref[:BN, :] = u[:, :DH]
    u_ref[BN:, :] = u[:, DH:]
    v_ref[...] = (
        lax.dot_general(xb, wr_ref[...], _DN, preferred_element_type=jnp.float32)
        + b_ref[...]
    )


def _tc_front(x, Wl, Wr, b):
    return pl.pallas_call(
        _front_body,
        grid=(NBLK,),
        in_specs=[
            pl.BlockSpec((BN, D), lambda i: (i, 0)),
            pl.BlockSpec((D, D), lambda i: (0, 0)),
            pl.BlockSpec((D, D), lambda i: (0, 0)),
            pl.BlockSpec((1, D), lambda i: (0, 0)),
        ],
        out_specs=[
            pl.BlockSpec((2 * BN, DH), lambda i: (i, 0)),
            pl.BlockSpec((BN, D), lambda i: (i, 0)),
        ],
        out_shape=[
            jax.ShapeDtypeStruct((2 * N, DH), jnp.float32),
            jax.ShapeDtypeStruct((N, D), jnp.float32),
        ],
    )(x, Wl.astype(jnp.bfloat16), Wr.astype(jnp.bfloat16), b.reshape(1, D))


def _mid_body(alo_ref, ahi_ref, c0_ref, c1_ref, v1_ref, wl_ref, wr_ref, b_ref, u_ref, v_ref):
    inv = 1.0 / jnp.maximum(c0_ref[:, 0:1] + c1_ref[:, 0:1], 1.0)
    h = jnp.concatenate([alo_ref[...], ahi_ref[...]], axis=1) * inv + v1_ref[...]
    h = jnp.maximum(h, 0.0).astype(jnp.bfloat16)
    u = lax.dot_general(h, wl_ref[...], _DN, preferred_element_type=jnp.float32)
    u_ref[:BN, :] = u[:, :DH]
    u_ref[BN:, :] = u[:, DH:]
    v_ref[...] = (
        lax.dot_general(h, wr_ref[...], _DN, preferred_element_type=jnp.float32)
        + b_ref[...]
    )


def _tc_mid(agg, cnt, v1, Wl, Wr, b):
    return pl.pallas_call(
        _mid_body,
        grid=(NBLK,),
        in_specs=[
            pl.BlockSpec((BN, DH), lambda i: (i, 0)),
            pl.BlockSpec((BN, DH), lambda i: (NBLK + i, 0)),
            pl.BlockSpec((BN, DH), lambda i: (i, 0)),
            pl.BlockSpec((BN, DH), lambda i: (NBLK + i, 0)),
            pl.BlockSpec((BN, D), lambda i: (i, 0)),
            pl.BlockSpec((D, D), lambda i: (0, 0)),
            pl.BlockSpec((D, D), lambda i: (0, 0)),
            pl.BlockSpec((1, D), lambda i: (0, 0)),
        ],
        out_specs=[
            pl.BlockSpec((2 * BN, DH), lambda i: (i, 0)),
            pl.BlockSpec((BN, D), lambda i: (i, 0)),
        ],
        out_shape=[
            jax.ShapeDtypeStruct((2 * N, DH), jnp.float32),
            jax.ShapeDtypeStruct((N, D), jnp.float32),
        ],
    )(agg, agg, cnt, cnt, v1, Wl.astype(jnp.bfloat16), Wr.astype(jnp.bfloat16), b.reshape(1, D))


def _out_body(alo_ref, ahi_ref, c0_ref, c1_ref, v2_ref, o_ref):
    inv = 1.0 / jnp.maximum(c0_ref[:, 0:1] + c1_ref[:, 0:1], 1.0)
    o_ref[...] = jnp.concatenate([alo_ref[...], ahi_ref[...]], axis=1) * inv + v2_ref[...]


def _tc_out(agg, cnt, v2):
    return pl.pallas_call(
        _out_body,
        grid=(NBLK,),
        in_specs=[
            pl.BlockSpec((BN, DH), lambda i: (i, 0)),
            pl.BlockSpec((BN, DH), lambda i: (NBLK + i, 0)),
            pl.BlockSpec((BN, DH), lambda i: (i, 0)),
            pl.BlockSpec((BN, DH), lambda i: (NBLK + i, 0)),
            pl.BlockSpec((BN, D), lambda i: (i, 0)),
        ],
        out_specs=pl.BlockSpec((BN, D), lambda i: (i, 0)),
        out_shape=jax.ShapeDtypeStruct((N, D), jnp.float32),
    )(agg, agg, cnt, cnt, v2)


NBUF = 2
NSUB = 1
SB = BATCH // NSUB
G = 16
NSEG = NCH // G


def _make_sc_agg():
    mesh = plsc.VectorSubcoreMesh(core_axis_name="c", subcore_axis_name="s")
    scratch = (
        pltpu.VMEM((2, G, BATCH), jnp.int32),
        pltpu.VMEM((2, G, BATCH), jnp.int32),
        pltpu.VMEM((NBUF, BATCH, DH), jnp.float32),
        pltpu.VMEM_SHARED((NACC, DH), jnp.float32),
        pltpu.SemaphoreType.DMA,
    ) + tuple(pltpu.SemaphoreType.DMA for _ in range(2 * NBUF))

    @functools.partial(
        pl.kernel,
        out_type=jax.ShapeDtypeStruct((2 * N, DH), jnp.float32),
        mesh=mesh,
        scratch_types=scratch,
    )
    def sc_agg(u_hbm, srcm_hbm, dstm_hbm, z128_hbm, agg_hbm,
               src_v, dst_v, rows_v, acc_sh, isem, *sems):
        gsem = sems[:NBUF]
        ssem = sems[NBUF:]
        c = lax.axis_index("c")
        s = lax.axis_index("s")
        rowsl = pl.ds(s * WPT, WPT)
        tail = pl.ds(NS * WPT, WTAIL)
        pltpu.sync_copy(z128_hbm, acc_sh.at[rowsl, :])

        @pl.when(s == NS - 1)
        def _():
            pltpu.sync_copy(z128_hbm.at[pl.ds(0, WTAIL), :], acc_sh.at[tail, :])

        splane = srcm_hbm.at[c * NS + s]
        dplane = dstm_hbm.at[s]

        def fire_seg(q, r):
            off = pl.multiple_of(q * G, G)
            pltpu.async_copy(splane.at[pl.ds(off, G), :], src_v.at[r], isem)
            pltpu.async_copy(dplane.at[pl.ds(off, G), :], dst_v.at[r], isem)

        def wait_seg(r):
            pltpu.make_async_copy(splane.at[pl.ds(0, G), :], src_v.at[r], isem).wait()
            pltpu.make_async_copy(dplane.at[pl.ds(0, G), :], dst_v.at[r], isem).wait()

        pltpu.sync_copy(splane.at[pl.ds(0, G), :], src_v.at[0])
        pltpu.sync_copy(dplane.at[pl.ds(0, G), :], dst_v.at[0])
        fire_seg(1, 1)
        plsc.subcore_barrier()

        def fire_g(r, t, b):
            for h in range(NSUB):
                pltpu.async_copy(
                    u_hbm.at[src_v.at[r, t, pl.ds(h * SB, SB)]],
                    rows_v.at[b, pl.ds(h * SB, SB), :],
                    gsem[b],
                )

        def wait_g(b):
            for h in range(NSUB):
                pltpu.make_async_copy(
                    u_hbm.at[src_v.at[0, 0, pl.ds(0, SB)]],
                    rows_v.at[b, pl.ds(0, SB), :],
                    gsem[b],
                ).wait()

        def fire_s(r, t, b):
            pltpu.async_copy(rows_v.at[b], acc_sh.at[dst_v.at[r, t]], ssem[b], add=True)

        def wait_s(b):
            pltpu.make_async_copy(rows_v.at[b], acc_sh.at[dst_v.at[0, 0]], ssem[b]).wait()

        def slot(r, rn, t, first_seg=False, last_seg=False, fire_next=None):
            b = t % 2
            wait_g(b)
            fire_s(r, t, b)
            if not (first_seg and t == 0):
                wait_s((t + 1) % 2)
            if t == 0 and fire_next is not None:
                fire_next()
            if t == G - 2 and not last_seg:
                wait_seg(rn)
            if not (last_seg and t == G - 1):
                tn = 0 if t == G - 1 else t + 1
                fire_g(rn if t == G - 1 else r, tn, (t + 1) % 2)

        fire_g(0, 0, 0)
        for t in range(G):
            slot(0, 1, t, first_seg=True)

        def steady(q, carry):
            r = lax.rem(q, 2)
            rn = lax.rem(q + 1, 2)

            def fire_next():
                fire_seg(q + 1, rn)

            for t in range(G):
                slot(r, rn, t, fire_next=fire_next)
            return carry

        lax.fori_loop(1, NSEG - 1, steady, 0)

        for t in range(G):
            slot(0, 1, t, last_seg=True)
        wait_s(1)
        plsc.subcore_barrier()
        pltpu.sync_copy(
            acc_sh.at[rowsl, :], agg_hbm.at[pl.ds(c * N + s * WPT, WPT), :]
        )

        @pl.when(s == NS - 1)
        def _():
            pltpu.sync_copy(
                acc_sh.at[tail, :], agg_hbm.at[pl.ds(c * N + NS * WPT, WTAIL), :]
            )

    return sc_agg


def _make_sc_cnt():
    mesh = plsc.VectorSubcoreMesh(core_axis_name="c", subcore_axis_name="s")
    scratch = (
        pltpu.VMEM((NCH, BATCH), jnp.int32),
        pltpu.VMEM((BATCH, DH), jnp.float32),
        pltpu.VMEM_SHARED((NACC, DH), jnp.float32),
    )

    @functools.partial(
        pl.kernel,
        out_type=jax.ShapeDtypeStruct((2 * N, DH), jnp.float32),
        mesh=mesh,
        scratch_types=scratch,
    )
    def sc_cnt(dstm_hbm, ones_hbm, z128_hbm, cnt_hbm, dst_v, ones_v, cnt_sh):
        c = lax.axis_index("c")
        s = lax.axis_index("s")
        rowsl = pl.ds(s * WPT, WPT)
        tail = pl.ds(NS * WPT, WTAIL)
        pltpu.sync_copy(z128_hbm, cnt_sh.at[rowsl, :])

        @pl.when(s == NS - 1)
        def _():
            pltpu.sync_copy(z128_hbm.at[pl.ds(0, WTAIL), :], cnt_sh.at[tail, :])

        pltpu.sync_copy(dstm_hbm.at[s], dst_v)
        pltpu.sync_copy(ones_hbm, ones_v)
        plsc.subcore_barrier()

        def step(j, carry):
            pltpu.sync_copy(ones_v, cnt_sh.at[dst_v.at[c * (NCH // 2) + j]], add=True)
            return carry

        lax.fori_loop(0, NCH // 2, step, 0)
        plsc.subcore_barrier()
        pltpu.sync_copy(
            cnt_sh.at[rowsl, :], cnt_hbm.at[pl.ds(c * N + s * WPT, WPT), :]
        )

        @pl.when(s == NS - 1)
        def _():
            pltpu.sync_copy(
                cnt_sh.at[tail, :], cnt_hbm.at[pl.ds(c * N + NS * WPT, WTAIL), :]
            )

    return sc_cnt


@functools.lru_cache(maxsize=None)
def _sc_get(which):
    return _make_sc_agg() if which == "agg" else _make_sc_cnt()


def _sc_agg(*args):
    return _sc_get("agg")(*args)


def _sc_cnt(*args):
    return _sc_get("cnt")(*args)


def kernel(x, edge_list, W1l, b1l, W1r, W2l, b2l, W2r):
    src = edge_list[0].astype(jnp.int32)
    dst = edge_list[1].astype(jnp.int32)
    blk = src // BN
    rem = src % BN
    src0 = blk * (2 * BN) + rem
    pad_src = jnp.zeros((NS, PADT), jnp.int32)
    pad_dst = jnp.full((NS, PADT), N, jnp.int32)

    def tile_pad(v, padv):
        return jnp.concatenate([v.reshape(NS, EPT), padv], axis=1).reshape(
            NS, NCH, BATCH
        )

    srcm = jnp.concatenate(
        [tile_pad(src0, pad_src), tile_pad(src0 + BN, pad_src)], axis=0
    )
    dstm = tile_pad(dst, pad_dst)
    ones = jnp.ones((BATCH, DH), jnp.float32)
    z128 = jnp.zeros((WPT, DH), jnp.float32)

    cnt = _sc_cnt(dstm, ones, z128)
    u1, v1 = _tc_front(x, W1l, W1r, b1l)
    agg1 = _sc_agg(u1, srcm, dstm, z128)
    u2, v2 = _tc_mid(agg1, cnt, v1, W2l, W2r, b2l)
    agg2 = _sc_agg(u2, srcm, dstm, z128)
    return _tc_out(agg2, cnt, v2)


# speedup vs baseline: 1.0033x; 1.0033x over previous
import functools

import jax
import jax.numpy as jnp
from jax import lax
from jax.experimental import pallas as pl
from jax.experimental.pallas import tpu as pltpu
from jax.experimental.pallas import tpu_sc as plsc

N = 10000
E = 160000
D = 256
DH = 128
NC = 2
NS = 16
BN = 400
NBLK = N // BN
EPT = E // NS
BATCH = 128
NCH = 80
SLOTS = NCH * BATCH
PADT = SLOTS - EPT
NJ = 16
NACC = N + NJ
WPT = 624
WTAIL = N - NS * WPT

_DN = (((1,), (1,)), ((), ()))


def _front_body(x_ref, wl_ref, wr_ref, b_ref, u_ref, v_ref):
    xb = x_ref[...]
    u = lax.dot_general(xb, wl_ref[...], _DN, preferred_element_type=jnp.float32)
    u_ref[:BN, :] = u[:, :DH]
    u_ref[BN:, :] = u[:, DH:]
    v_ref[...] = (
        lax.dot_general(xb, wr_ref[...], _DN, preferred_element_type=jnp.float32)
        + b_ref[...]
    )


def _tc_front(x, Wl, Wr, b):
    return pl.pallas_call(
        _front_body,
        grid=(NBLK,),
        in_specs=[
            pl.BlockSpec((BN, D), lambda i: (i, 0)),
            pl.BlockSpec((D, D), lambda i: (0, 0)),
            pl.BlockSpec((D, D), lambda i: (0, 0)),
            pl.BlockSpec((1, D), lambda i: (0, 0)),
        ],
        out_specs=[
            pl.BlockSpec((2 * BN, DH), lambda i: (i, 0)),
            pl.BlockSpec((BN, D), lambda i: (i, 0)),
        ],
        out_shape=[
            jax.ShapeDtypeStruct((2 * N, DH), jnp.float32),
            jax.ShapeDtypeStruct((N, D), jnp.float32),
        ],
    )(x, Wl, Wr, b.reshape(1, D))


def _mid_body(alo_ref, ahi_ref, c0_ref, c1_ref, v1_ref, wl_ref, wr_ref, b_ref, u_ref, v_ref):
    inv = 1.0 / jnp.maximum(c0_ref[:, 0:1] + c1_ref[:, 0:1], 1.0)
    h = jnp.concatenate([alo_ref[...], ahi_ref[...]], axis=1) * inv + v1_ref[...]
    h = jnp.maximum(h, 0.0)
    u = lax.dot_general(h, wl_ref[...], _DN, preferred_element_type=jnp.float32)
    u_ref[:BN, :] = u[:, :DH]
    u_ref[BN:, :] = u[:, DH:]
    v_ref[...] = (
        lax.dot_general(h, wr_ref[...], _DN, preferred_element_type=jnp.float32)
        + b_ref[...]
    )


def _tc_mid(agg, cnt, v1, Wl, Wr, b):
    return pl.pallas_call(
        _mid_body,
        grid=(NBLK,),
        in_specs=[
            pl.BlockSpec((BN, DH), lambda i: (i, 0)),
            pl.BlockSpec((BN, DH), lambda i: (NBLK + i, 0)),
            pl.BlockSpec((BN, DH), lambda i: (i, 0)),
            pl.BlockSpec((BN, DH), lambda i: (NBLK + i, 0)),
            pl.BlockSpec((BN, D), lambda i: (i, 0)),
            pl.BlockSpec((D, D), lambda i: (0, 0)),
            pl.BlockSpec((D, D), lambda i: (0, 0)),
            pl.BlockSpec((1, D), lambda i: (0, 0)),
        ],
        out_specs=[
            pl.BlockSpec((2 * BN, DH), lambda i: (i, 0)),
            pl.BlockSpec((BN, D), lambda i: (i, 0)),
        ],
        out_shape=[
            jax.ShapeDtypeStruct((2 * N, DH), jnp.float32),
            jax.ShapeDtypeStruct((N, D), jnp.float32),
        ],
    )(agg, agg, cnt, cnt, v1, Wl, Wr, b.reshape(1, D))


def _out_body(alo_ref, ahi_ref, c0_ref, c1_ref, v2_ref, o_ref):
    inv = 1.0 / jnp.maximum(c0_ref[:, 0:1] + c1_ref[:, 0:1], 1.0)
    o_ref[...] = jnp.concatenate([alo_ref[...], ahi_ref[...]], axis=1) * inv + v2_ref[...]


def _tc_out(agg, cnt, v2):
    return pl.pallas_call(
        _out_body,
        grid=(NBLK,),
        in_specs=[
            pl.BlockSpec((BN, DH), lambda i: (i, 0)),
            pl.BlockSpec((BN, DH), lambda i: (NBLK + i, 0)),
            pl.BlockSpec((BN, DH), lambda i: (i, 0)),
            pl.BlockSpec((BN, DH), lambda i: (NBLK + i, 0)),
            pl.BlockSpec((BN, D), lambda i: (i, 0)),
        ],
        out_specs=pl.BlockSpec((BN, D), lambda i: (i, 0)),
        out_shape=jax.ShapeDtypeStruct((N, D), jnp.float32),
    )(agg, agg, cnt, cnt, v2)


NBUF = 2
NSUB = 1
SB = BATCH // NSUB
G = 16
NSEG = NCH // G


def _make_sc_agg():
    mesh = plsc.VectorSubcoreMesh(core_axis_name="c", subcore_axis_name="s")
    scratch = (
        pltpu.VMEM((2, G, BATCH), jnp.int32),
        pltpu.VMEM((2, G, BATCH), jnp.int32),
        pltpu.VMEM((NBUF, BATCH, DH), jnp.float32),
        pltpu.VMEM_SHARED((NACC, DH), jnp.float32),
        pltpu.SemaphoreType.DMA,
    ) + tuple(pltpu.SemaphoreType.DMA for _ in range(2 * NBUF))

    @functools.partial(
        pl.kernel,
        out_type=jax.ShapeDtypeStruct((2 * N, DH), jnp.float32),
        mesh=mesh,
        scratch_types=scratch,
    )
    def sc_agg(u_hbm, srcm_hbm, dstm_hbm, z128_hbm, agg_hbm,
               src_v, dst_v, rows_v, acc_sh, isem, *sems):
        gsem = sems[:NBUF]
        ssem = sems[NBUF:]
        c = lax.axis_index("c")
        s = lax.axis_index("s")
        rowsl = pl.ds(s * WPT, WPT)
        tail = pl.ds(NS * WPT, WTAIL)
        pltpu.sync_copy(z128_hbm, acc_sh.at[rowsl, :])

        @pl.when(s == NS - 1)
        def _():
            pltpu.sync_copy(z128_hbm.at[pl.ds(0, WTAIL), :], acc_sh.at[tail, :])

        splane = srcm_hbm.at[c * NS + s]
        dplane = dstm_hbm.at[s]

        def fire_seg(q, r):
            off = pl.multiple_of(q * G, G)
            pltpu.async_copy(splane.at[pl.ds(off, G), :], src_v.at[r], isem)
            pltpu.async_copy(dplane.at[pl.ds(off, G), :], dst_v.at[r], isem)

        def wait_seg(r):
            pltpu.make_async_copy(splane.at[pl.ds(0, G), :], src_v.at[r], isem).wait()
            pltpu.make_async_copy(dplane.at[pl.ds(0, G), :], dst_v.at[r], isem).wait()

        pltpu.sync_copy(splane.at[pl.ds(0, G), :], src_v.at[0])
        pltpu.sync_copy(dplane.at[pl.ds(0, G), :], dst_v.at[0])
        fire_seg(1, 1)
        plsc.subcore_barrier()

        def fire_g(r, t, b):
            for h in range(NSUB):
                pltpu.async_copy(
                    u_hbm.at[src_v.at[r, t, pl.ds(h * SB, SB)]],
                    rows_v.at[b, pl.ds(h * SB, SB), :],
                    gsem[b],
                )

        def wait_g(b):
            for h in range(NSUB):
                pltpu.make_async_copy(
                    u_hbm.at[src_v.at[0, 0, pl.ds(0, SB)]],
                    rows_v.at[b, pl.ds(0, SB), :],
                    gsem[b],
                ).wait()

        def fire_s(r, t, b):
            pltpu.async_copy(rows_v.at[b], acc_sh.at[dst_v.at[r, t]], ssem[b], add=True)

        def wait_s(b):
            pltpu.make_async_copy(rows_v.at[b], acc_sh.at[dst_v.at[0, 0]], ssem[b]).wait()

        def slot(r, rn, t, first_seg=False, last_seg=False, fire_next=None):
            b = t % 2
            wait_g(b)
            fire_s(r, t, b)
            if not (first_seg and t == 0):
                wait_s((t + 1) % 2)
            if t == 0 and fire_next is not None:
                fire_next()
            if t == G - 2 and not last_seg:
                wait_seg(rn)
            if not (last_seg and t == G - 1):
                tn = 0 if t == G - 1 else t + 1
                fire_g(rn if t == G - 1 else r, tn, (t + 1) % 2)

        fire_g(0, 0, 0)
        for t in range(G):
            slot(0, 1, t, first_seg=True)

        def steady(q, carry):
            r = lax.rem(q, 2)
            rn = lax.rem(q + 1, 2)

            def fire_next():
                fire_seg(q + 1, rn)

            for t in range(G):
                slot(r, rn, t, fire_next=fire_next)
            return carry

        lax.fori_loop(1, NSEG - 1, steady, 0)

        for t in range(G):
            slot(0, 1, t, last_seg=True)
        wait_s(1)
        plsc.subcore_barrier()
        pltpu.sync_copy(
            acc_sh.at[rowsl, :], agg_hbm.at[pl.ds(c * N + s * WPT, WPT), :]
        )

        @pl.when(s == NS - 1)
        def _():
            pltpu.sync_copy(
                acc_sh.at[tail, :], agg_hbm.at[pl.ds(c * N + NS * WPT, WTAIL), :]
            )

    return sc_agg


def _make_sc_cnt():
    mesh = plsc.VectorSubcoreMesh(core_axis_name="c", subcore_axis_name="s")
    scratch = (
        pltpu.VMEM((NCH, BATCH), jnp.int32),
        pltpu.VMEM((BATCH, DH), jnp.float32),
        pltpu.VMEM_SHARED((NACC, DH), jnp.float32),
    )

    @functools.partial(
        pl.kernel,
        out_type=jax.ShapeDtypeStruct((2 * N, DH), jnp.float32),
        mesh=mesh,
        scratch_types=scratch,
    )
    def sc_cnt(dstm_hbm, ones_hbm, z128_hbm, cnt_hbm, dst_v, ones_v, cnt_sh):
        c = lax.axis_index("c")
        s = lax.axis_index("s")
        rowsl = pl.ds(s * WPT, WPT)
        tail = pl.ds(NS * WPT, WTAIL)
        pltpu.sync_copy(z128_hbm, cnt_sh.at[rowsl, :])

        @pl.when(s == NS - 1)
        def _():
            pltpu.sync_copy(z128_hbm.at[pl.ds(0, WTAIL), :], cnt_sh.at[tail, :])

        pltpu.sync_copy(dstm_hbm.at[s], dst_v)
        pltpu.sync_copy(ones_hbm, ones_v)
        plsc.subcore_barrier()

        def step(j, carry):
            pltpu.sync_copy(ones_v, cnt_sh.at[dst_v.at[c * (NCH // 2) + j]], add=True)
            return carry

        lax.fori_loop(0, NCH // 2, step, 0)
        plsc.subcore_barrier()
        pltpu.sync_copy(
            cnt_sh.at[rowsl, :], cnt_hbm.at[pl.ds(c * N + s * WPT, WPT), :]
        )

        @pl.when(s == NS - 1)
        def _():
            pltpu.sync_copy(
                cnt_sh.at[tail, :], cnt_hbm.at[pl.ds(c * N + NS * WPT, WTAIL), :]
            )

    return sc_cnt


@functools.lru_cache(maxsize=None)
def _sc_get(which):
    return _make_sc_agg() if which == "agg" else _make_sc_cnt()


def _sc_agg(*args):
    return _sc_get("agg")(*args)


def _sc_cnt(*args):
    return _sc_get("cnt")(*args)


def kernel(x, edge_list, W1l, b1l, W1r, W2l, b2l, W2r):
    src = edge_list[0].astype(jnp.int32)
    dst = edge_list[1].astype(jnp.int32)
    blk = src // BN
    rem = src % BN
    src0 = blk * (2 * BN) + rem
    pad_src = jnp.zeros((NS, PADT), jnp.int32)
    pad_dst = jnp.full((NS, PADT), N, jnp.int32)

    def tile_pad(v, padv):
        return jnp.concatenate([v.reshape(NS, EPT), padv], axis=1).reshape(
            NS, NCH, BATCH
        )

    srcm = jnp.concatenate(
        [tile_pad(src0, pad_src), tile_pad(src0 + BN, pad_src)], axis=0
    )
    dstm = tile_pad(dst, pad_dst)
    ones = jnp.ones((BATCH, DH), jnp.float32)
    z128 = jnp.zeros((WPT, DH), jnp.float32)

    cnt = _sc_cnt(dstm, ones, z128)
    u1, v1 = _tc_front(x, W1l, W1r, b1l)
    agg1 = _sc_agg(u1, srcm, dstm, z128)
    u2, v2 = _tc_mid(agg1, cnt, v1, W2l, W2r, b2l)
    agg2 = _sc_agg(u2, srcm, dstm, z128)
    return _tc_out(agg2, cnt, v2)
